# initial kernel scaffold (unmeasured)
import jax
import jax.numpy as jnp
from jax import lax
from jax.experimental import pallas as pl
from jax.experimental.pallas import tpu as pltpu

N_DEV = 32
N_FWD = N_DEV // 2
N_BWD = N_DEV - 1 - N_FWD


def _gelu(y):
    c = 0.7978845608028654
    return 0.5 * y * (1.0 + jnp.tanh(c * (y + 0.044715 * y * y * y)))


def kernel(x, w_mat):
    m_per, k = x.shape
    _, n_per = w_mat.shape

    x = x.astype(jnp.bfloat16)
    w_mat = w_mat.astype(jnp.bfloat16)

    def body(x_ref, w_ref, out_ref, fwd_buf, bwd_buf,
             fwd_send, fwd_recv, bwd_send, bwd_recv):
        my = lax.axis_index("i")
        left = (my - 1) % N_DEV
        right = (my + 1) % N_DEV

        barrier = pltpu.get_barrier_semaphore()
        for nbr in (left, right):
            pl.semaphore_signal(
                barrier, inc=1,
                device_id=(nbr,), device_id_type=pl.DeviceIdType.MESH,
            )
        pl.semaphore_wait(barrier, 2)

        def tile(origin, chunk):
            out_ref[pl.ds(origin * m_per, m_per), :] = _gelu(
                jnp.dot(chunk, w_ref[...], preferred_element_type=jnp.float32)
            )

        tile(my, x_ref[...])

        for h in range(N_FWD):
            slot = h % 2
            fwd = pltpu.make_async_remote_copy(
                src_ref=x_ref if h == 0 else fwd_buf.at[(h - 1) % 2],
                dst_ref=fwd_buf.at[slot],
                send_sem=fwd_send.at[slot],
                recv_sem=fwd_recv.at[slot],
                device_id=(right,),
                device_id_type=pl.DeviceIdType.MESH,
            )
            fwd.start()
            bwd = None
            if h < N_BWD:
                bwd = pltpu.make_async_remote_copy(
                    src_ref=x_ref if h == 0 else bwd_buf.at[(h - 1) % 2],
                    dst_ref=bwd_buf.at[slot],
                    send_sem=bwd_send.at[slot],
                    recv_sem=bwd_recv.at[slot],
                    device_id=(left,),
                    device_id_type=pl.DeviceIdType.MESH,
                )
                bwd.start()
            fwd.wait()
            if bwd is not None:
                bwd.wait()
            tile((my - 1 - h) % N_DEV, fwd_buf[slot])
            if bwd is not None:
                tile((my + 1 + h) % N_DEV, bwd_buf[slot])

    return pl.pallas_call(
        body,
        out_shape=jax.ShapeDtypeStruct((N_DEV * m_per, n_per), jnp.float32),
        in_specs=[
            pl.BlockSpec(memory_space=pltpu.VMEM),
            pl.BlockSpec(memory_space=pltpu.VMEM),
        ],
        out_specs=pl.BlockSpec(memory_space=pltpu.VMEM),
        scratch_shapes=[
            pltpu.VMEM((2, m_per, k), jnp.bfloat16),
            pltpu.VMEM((2, m_per, k), jnp.bfloat16),
            pltpu.SemaphoreType.DMA((2,)),
            pltpu.SemaphoreType.DMA((2,)),
            pltpu.SemaphoreType.DMA((2,)),
            pltpu.SemaphoreType.DMA((2,)),
        ],
        compiler_params=pltpu.CompilerParams(collective_id=0),
    )(x, w_mat)


# baseline (device time: 766362 ns/iter reference)
import jax
import jax.numpy as jnp
import numpy as np
from jax import lax
from jax.experimental import pallas as pl
from jax.experimental.pallas import tpu as pltpu

N_DEV = 32
N_FWD = N_DEV // 2
N_BWD = N_DEV - 1 - N_FWD
N_SLOT = 4


def _logical_id(x, y, z):
    return z * 8 + y * 2 + (x if y % 2 == 0 else 1 - x)


def _ham_cycle():
    path_yz = []
    for z in range(4):
        ys = range(4) if z % 2 == 0 else range(3, -1, -1)
        path_yz.extend((y, z) for y in ys)
    cycle = [(0, y, z) for y, z in path_yz]
    cycle += [(1, y, z) for y, z in reversed(path_yz)]
    return [_logical_id(*c) for c in cycle]


_HAM = np.array(_ham_cycle(), dtype=np.int32)
_POS = np.argsort(_HAM).astype(np.int32)


def _gelu(y):
    c = 0.7978845608028654
    return 0.5 * y * (1.0 + jnp.tanh(c * (y + 0.044715 * y * y * y)))


def kernel(x, w_mat):
    m_per, k = x.shape
    _, n_per = w_mat.shape

    x = x.astype(jnp.bfloat16)
    w_mat = w_mat.astype(jnp.bfloat16)

    ham = jnp.asarray(_HAM)
    pos = jnp.asarray(_POS)[lax.axis_index("i")]
    right = ham[(pos + 1) % N_DEV]
    left = ham[(pos - 1) % N_DEV]
    orig_fwd = ham[(pos - 1 - jnp.arange(N_FWD)) % N_DEV]
    orig_bwd = ham[(pos + 1 + jnp.arange(N_BWD)) % N_DEV]
    meta = jnp.concatenate(
        [jnp.stack([right, left]), orig_fwd, orig_bwd]
    ).astype(jnp.int32)

    def body(meta_ref, x_ref, w_ref, out_ref, fwd_buf, bwd_buf,
             fwd_send, fwd_recv, bwd_send, bwd_recv,
             fwd_credit, bwd_credit):
        right_id = meta_ref[0]
        left_id = meta_ref[1]

        barrier = pltpu.get_barrier_semaphore()
        for nbr in (left_id, right_id):
            pl.semaphore_signal(
                barrier, inc=1,
                device_id=(nbr,), device_id_type=pl.DeviceIdType.MESH,
            )
        pl.semaphore_wait(barrier, 2)

        def tile(origin, chunk):
            out_ref[pl.ds(origin * m_per, m_per), :] = _gelu(
                jnp.dot(chunk, w_ref[...], preferred_element_type=jnp.float32)
            )

        def mk(src, buf, send_sems, recv_sems, slot, dst_id):
            return pltpu.make_async_remote_copy(
                src_ref=src,
                dst_ref=buf.at[slot],
                send_sem=send_sems.at[slot],
                recv_sem=recv_sems.at[slot],
                device_id=(dst_id,),
                device_id_type=pl.DeviceIdType.MESH,
            )

        fw = mk(x_ref, fwd_buf, fwd_send, fwd_recv, 0, right_id)
        fw.start()
        bw = mk(x_ref, bwd_buf, bwd_send, bwd_recv, 0, left_id)
        bw.start()

        tile(lax.axis_index("i"), x_ref[...])

        for h in range(N_FWD):
            slot = h % N_SLOT
            fw.wait_recv()
            fw_next = None
            if h + 1 < N_FWD:
                if h + 1 >= N_SLOT:
                    pl.semaphore_wait(fwd_credit, 1)
                fw_next = mk(fwd_buf.at[slot], fwd_buf, fwd_send, fwd_recv,
                             (h + 1) % N_SLOT, right_id)
                fw_next.start()
            fw.wait_send()
            if 1 <= h <= N_FWD - N_SLOT:
                pl.semaphore_signal(
                    fwd_credit, inc=1,
                    device_id=(left_id,), device_id_type=pl.DeviceIdType.MESH,
                )
            bw_next = None
            if h < N_BWD:
                bw.wait_recv()
                if h + 1 < N_BWD:
                    if h + 1 >= N_SLOT:
                        pl.semaphore_wait(bwd_credit, 1)
                    bw_next = mk(bwd_buf.at[slot], bwd_buf, bwd_send,
                                 bwd_recv, (h + 1) % N_SLOT, left_id)
                    bw_next.start()
                bw.wait_send()
                if 1 <= h <= N_BWD - N_SLOT:
                    pl.semaphore_signal(
                        bwd_credit, inc=1,
                        device_id=(right_id,),
                        device_id_type=pl.DeviceIdType.MESH,
                    )
            tile(meta_ref[2 + h], fwd_buf[slot])
            if h < N_BWD:
                tile(meta_ref[2 + N_FWD + h], bwd_buf[slot])
            fw = fw_next
            bw = bw_next

    return pl.pallas_call(
        body,
        out_shape=jax.ShapeDtypeStruct((N_DEV * m_per, n_per), jnp.float32),
        in_specs=[
            pl.BlockSpec(memory_space=pltpu.SMEM),
            pl.BlockSpec(memory_space=pltpu.VMEM),
            pl.BlockSpec(memory_space=pltpu.VMEM),
        ],
        out_specs=pl.BlockSpec(memory_space=pltpu.VMEM),
        scratch_shapes=[
            pltpu.VMEM((N_SLOT, m_per, k), jnp.bfloat16),
            pltpu.VMEM((N_SLOT, m_per, k), jnp.bfloat16),
            pltpu.SemaphoreType.DMA((N_SLOT,)),
            pltpu.SemaphoreType.DMA((N_SLOT,)),
            pltpu.SemaphoreType.DMA((N_SLOT,)),
            pltpu.SemaphoreType.DMA((N_SLOT,)),
            pltpu.SemaphoreType.REGULAR,
            pltpu.SemaphoreType.REGULAR,
        ],
        compiler_params=pltpu.CompilerParams(collective_id=0),
    )(meta, x, w_mat)
